# Initial kernel scaffold; baseline (speedup 1.0000x reference)
import jax
import jax.numpy as jnp
from jax.experimental import pallas as pl


def kernel(input, target, labels):
    raise NotImplementedError("write your pallas kernel here")


# speedup vs baseline: 150.9012x; 150.9012x over previous
import functools

import jax
import jax.numpy as jnp
import numpy as np
from jax import lax
from jax.experimental import pallas as pl
from jax.experimental.pallas import tpu as pltpu
from jax.experimental.pallas import tpu_sc as plsc

_L = 16
_NS = 16
_NL = 100000
_NB = 16384
_LCH = 6256
_NPAD = _NS * _LCH
_TCH = _NB // _NS
_KS = 9
_SIGMA = 1.0

_half = (_KS - 1) // 2
_x = np.arange(-_half, _half + 1, dtype=np.float64)
_kw = np.exp(-0.5 * (_x / _SIGMA) ** 2)
_KW = [float(v) for v in (_kw / _kw.max()).astype(np.float32)]


def _sc_body(labels_hbm, input_hbm, target_hbm, out_hbm,
             lab_v, inp_v, tgt_v, histflat_v, hist_v, histpad_v, sd_v,
             edges_v, stage_v, tmpa_v, tmpb_v, tmph_v, outrow_v,
             mm_min_sh, mm_max_sh, hist_sh):
    cid = lax.axis_index("c")
    sid = lax.axis_index("s")
    lanes = lax.broadcasted_iota(jnp.int32, (_L,), 0)

    def _xlane(v, op):
        s = v[0]
        for r in range(1, _L):
            s = op(s, v[r])
        return jnp.full((_L,), s)

    zeros = jnp.zeros((_L,), jnp.float32)
    ones = jnp.ones((_L,), jnp.float32)
    inf = jnp.full((_L,), jnp.inf, jnp.float32)
    ninf = jnp.full((_L,), -jnp.inf, jnp.float32)

    base = sid * _LCH
    pltpu.sync_copy(labels_hbm.at[pl.ds(base, _LCH)], lab_v)

    def mm_body(i, carry):
        mn, mx = carry
        v = lab_v[pl.ds(i * _L, _L)]
        valid = (base + i * _L + lanes) < _NL
        mn = jnp.minimum(mn, jnp.where(valid, v, inf))
        mx = jnp.maximum(mx, jnp.where(valid, v, ninf))
        return mn, mx

    mn, mx = lax.fori_loop(0, _LCH // _L, mm_body, (inf, ninf))
    stage_v[0, :] = _xlane(mn, jnp.minimum)
    stage_v[1, :] = _xlane(mx, jnp.maximum)
    pltpu.sync_copy(stage_v.at[pl.ds(0, 1)], mm_min_sh.at[pl.ds(sid, 1)])
    pltpu.sync_copy(stage_v.at[pl.ds(1, 1)], mm_max_sh.at[pl.ds(sid, 1)])
    plsc.subcore_barrier()

    pltpu.sync_copy(mm_min_sh, tmpa_v)
    pltpu.sync_copy(mm_max_sh, tmpb_v)
    gmin = tmpa_v[0, :]
    gmax = tmpb_v[0, :]
    for r in range(1, _NS):
        gmin = jnp.minimum(gmin, tmpa_v[r, :])
        gmax = jnp.maximum(gmax, tmpb_v[r, :])

    nbi = ((gmax - gmin) / jnp.float32(0.1)).astype(jnp.int32)
    nbf = nbi.astype(jnp.float32)
    invr = nbf / (gmax - gmin)

    for j in range(9):
        jv = j * _L + lanes
        step = jv.astype(jnp.float32) / nbf
        e = gmin * (1.0 - step) + gmax * step
        e = jnp.where(jv == nbi, gmax, e)
        e = jnp.where(jv > nbi, inf, e)
        edges_v[pl.ds(j * _L, _L)] = e

    def bidx(v):
        c = jnp.clip(((v - gmin) * invr).astype(jnp.int32), 0, nbi - 1)
        e0 = plsc.load_gather(edges_v, [c])
        e1 = plsc.load_gather(edges_v, [c + 1])
        t = c - (e0 > v).astype(jnp.int32) + (e1 <= v).astype(jnp.int32)
        return jnp.where(t < 0, nbi - 1, jnp.minimum(t, nbi - 1))

    def zero_body(i, _):
        histflat_v[pl.ds(i * _L, _L)] = zeros
        return 0

    lax.fori_loop(0, (_NS * 128) // _L, zero_body, 0)

    def hist_body(i, _):
        v = lab_v[pl.ds(i * _L, _L)]
        valid = (base + i * _L + lanes) < _NL
        idx = bidx(v)
        plsc.addupdate_scatter(histflat_v, [lanes * 128 + idx], ones,
                               mask=valid)
        return 0

    lax.fori_loop(0, _LCH // _L, hist_body, 0)

    for j in range(8):
        acc = histflat_v[pl.ds(j * _L, _L)]
        for r in range(1, _NS):
            acc = acc + histflat_v[pl.ds(r * 128 + j * _L, _L)]
        hist_v[0, pl.ds(j * _L, _L)] = acc
    pltpu.sync_copy(hist_v, hist_sh.at[pl.ds(sid, 1)])
    plsc.subcore_barrier()

    pltpu.sync_copy(hist_sh, tmph_v)
    for j in range(10):
        histpad_v[pl.ds(j * _L, _L)] = zeros
    for j in range(8):
        acc = tmph_v[0, pl.ds(j * _L, _L)]
        for r in range(1, _NS):
            acc = acc + tmph_v[r, pl.ds(j * _L, _L)]
        histpad_v[pl.ds(_L + j * _L, _L)] = acc

    for j in range(8):
        s = zeros
        for k in range(_KS):
            tap = plsc.load_gather(
                histpad_v, [_L + j * _L + (k - _half) + lanes])
            s = s + jnp.float32(_KW[k]) * tap
        sd_v[pl.ds(j * _L, _L)] = s

    acc = zeros
    for j in range(8):
        h = histpad_v[pl.ds(_L + j * _L, _L)]
        s = sd_v[pl.ds(j * _L, _L)]
        acc = acc + jnp.where(h > 0.0, h / s, 0.0)
    scaling = jnp.float32(_NL) / _xlane(acc, jnp.add)

    tb = sid * _TCH
    pltpu.sync_copy(input_hbm.at[pl.ds(tb, _TCH)], inp_v)
    pltpu.sync_copy(target_hbm.at[pl.ds(tb, _TCH)], tgt_v)

    def tgt_body(i, carry):
        acc_a, acc_b, acc_w = carry
        tv = tgt_v[pl.ds(i * _L, _L)]
        iv = inp_v[pl.ds(i * _L, _L)]
        kw = plsc.load_gather(sd_v, [bidx(tv)])
        isz = kw == 0.0
        w = jnp.where(isz, zeros, 1.0 / kw)
        d = iv - tv
        loss = d * d
        acc_a = acc_a + jnp.where(isz, zeros, loss * w)
        acc_b = acc_b + jnp.where(isz, loss, zeros)
        acc_w = jnp.maximum(acc_w, jnp.where(isz, ninf, w))
        return acc_a, acc_b, acc_w

    acc_a, acc_b, acc_w = lax.fori_loop(
        0, _TCH // _L, tgt_body, (zeros, zeros, ninf))

    xa = _xlane(acc_a, jnp.add)
    xb = _xlane(acc_b, jnp.add)
    xw = _xlane(acc_w, jnp.maximum)
    vals = jnp.where(lanes == 0, xa,
                     jnp.where(lanes == 1, xb,
                               jnp.where(lanes == 2, xw,
                                         jnp.where(lanes == 3, scaling,
                                                   zeros))))
    outrow_v[0, :] = vals

    @pl.when(cid == 0)
    def _():
        pltpu.sync_copy(outrow_v, out_hbm.at[pl.ds(sid, 1)])


@jax.jit
def _ldsloss(labels_pad, inp, tgt):
    mesh = plsc.VectorSubcoreMesh(core_axis_name="c", subcore_axis_name="s",
                                  num_cores=2, num_subcores=_NS)
    f = pl.kernel(
        _sc_body,
        out_type=jax.ShapeDtypeStruct((_NS, _L), jnp.float32),
        mesh=mesh,
        compiler_params=pltpu.CompilerParams(needs_layout_passes=False),
        scratch_types=[
            pltpu.VMEM((_LCH,), jnp.float32),
            pltpu.VMEM((_TCH,), jnp.float32),
            pltpu.VMEM((_TCH,), jnp.float32),
            pltpu.VMEM((_NS * 128,), jnp.float32),
            pltpu.VMEM((1, 128), jnp.float32),
            pltpu.VMEM((160,), jnp.float32),
            pltpu.VMEM((128,), jnp.float32),
            pltpu.VMEM((144,), jnp.float32),
            pltpu.VMEM((2, _L), jnp.float32),
            pltpu.VMEM((_NS, _L), jnp.float32),
            pltpu.VMEM((_NS, _L), jnp.float32),
            pltpu.VMEM((_NS, 128), jnp.float32),
            pltpu.VMEM((1, _L), jnp.float32),
            pltpu.VMEM_SHARED((_NS, _L), jnp.float32),
            pltpu.VMEM_SHARED((_NS, _L), jnp.float32),
            pltpu.VMEM_SHARED((_NS, 128), jnp.float32),
        ],
    )
    rows = f(labels_pad, inp, tgt)
    av = jnp.sum(rows[:, 0])
    bv = jnp.sum(rows[:, 1])
    wv = jnp.max(rows[:, 2])
    scaling = rows[0, 3]
    wsafe = jnp.where(bv > 0.0, wv, jnp.float32(0.0))
    return (av + wsafe * bv) * scaling / jnp.float32(_NB)


def kernel(input, target, labels):
    labs = jnp.concatenate(
        [labels[:, 0], jnp.zeros((_NPAD - _NL,), jnp.float32)])
    return _ldsloss(labs, input[:, 0], target[:, 0])


# speedup vs baseline: 151.4603x; 1.0037x over previous
import functools

import jax
import jax.numpy as jnp
import numpy as np
from jax import lax
from jax.experimental import pallas as pl
from jax.experimental.pallas import tpu as pltpu
from jax.experimental.pallas import tpu_sc as plsc

_L = 16
_NS = 16
_NL = 100000
_NB = 16384
_LCH = 6256
_NPAD = _NS * _LCH
_TCH = _NB // _NS
_TC2 = _NB // (2 * _NS)
_KS = 9
_SIGMA = 1.0

_half = (_KS - 1) // 2
_x = np.arange(-_half, _half + 1, dtype=np.float64)
_kw = np.exp(-0.5 * (_x / _SIGMA) ** 2)
_KW = [float(v) for v in (_kw / _kw.max()).astype(np.float32)]


def _sc_body(labels_hbm, input_hbm, target_hbm, out_hbm,
             lab_v, inp_v, tgt_v, histflat_v, hist_v, histpad_v, sd_v,
             edges_v, stage_v, tmpa_v, tmpb_v, tmph_v, outrow_v,
             mm_min_sh, mm_max_sh, hist_sh):
    cid = lax.axis_index("c")
    sid = lax.axis_index("s")
    lanes = lax.broadcasted_iota(jnp.int32, (_L,), 0)

    def _xlane(v, op):
        s = v[0]
        for r in range(1, _L):
            s = op(s, v[r])
        return jnp.full((_L,), s)

    zeros = jnp.zeros((_L,), jnp.float32)
    ones = jnp.ones((_L,), jnp.float32)
    inf = jnp.full((_L,), jnp.inf, jnp.float32)
    ninf = jnp.full((_L,), -jnp.inf, jnp.float32)

    base = sid * _LCH
    pltpu.sync_copy(labels_hbm.at[pl.ds(base, _LCH)], lab_v)

    def mm_body(i, carry):
        mn, mx = carry
        v = lab_v[pl.ds(i * _L, _L)]
        valid = (base + i * _L + lanes) < _NL
        mn = jnp.minimum(mn, jnp.where(valid, v, inf))
        mx = jnp.maximum(mx, jnp.where(valid, v, ninf))
        return mn, mx

    mn, mx = lax.fori_loop(0, _LCH // _L, mm_body, (inf, ninf))
    stage_v[0, :] = _xlane(mn, jnp.minimum)
    stage_v[1, :] = _xlane(mx, jnp.maximum)
    pltpu.sync_copy(stage_v.at[pl.ds(0, 1)], mm_min_sh.at[pl.ds(sid, 1)])
    pltpu.sync_copy(stage_v.at[pl.ds(1, 1)], mm_max_sh.at[pl.ds(sid, 1)])
    plsc.subcore_barrier()

    pltpu.sync_copy(mm_min_sh, tmpa_v)
    pltpu.sync_copy(mm_max_sh, tmpb_v)
    gmin = tmpa_v[0, :]
    gmax = tmpb_v[0, :]
    for r in range(1, _NS):
        gmin = jnp.minimum(gmin, tmpa_v[r, :])
        gmax = jnp.maximum(gmax, tmpb_v[r, :])

    nbi = ((gmax - gmin) / jnp.float32(0.1)).astype(jnp.int32)
    nbf = nbi.astype(jnp.float32)
    invr = nbf / (gmax - gmin)

    for j in range(9):
        jv = j * _L + lanes
        step = jv.astype(jnp.float32) / nbf
        e = gmin * (1.0 - step) + gmax * step
        e = jnp.where(jv == nbi, gmax, e)
        e = jnp.where(jv > nbi, inf, e)
        edges_v[pl.ds(j * _L, _L)] = e

    def bidx(v):
        c = jnp.clip(((v - gmin) * invr).astype(jnp.int32), 0, nbi - 1)
        e0 = plsc.load_gather(edges_v, [c])
        e1 = plsc.load_gather(edges_v, [c + 1])
        t = c - (e0 > v).astype(jnp.int32) + (e1 <= v).astype(jnp.int32)
        return jnp.where(t < 0, nbi - 1, jnp.minimum(t, nbi - 1))

    def zero_body(i, _):
        histflat_v[pl.ds(i * _L, _L)] = zeros
        return 0

    lax.fori_loop(0, (_NS * 128) // _L, zero_body, 0)

    def hist_body(i, _):
        v = lab_v[pl.ds(i * _L, _L)]
        valid = (base + i * _L + lanes) < _NL
        idx = bidx(v)
        plsc.addupdate_scatter(histflat_v, [lanes * 128 + idx], ones,
                               mask=valid)
        return 0

    lax.fori_loop(0, _LCH // _L, hist_body, 0)

    for j in range(8):
        acc = histflat_v[pl.ds(j * _L, _L)]
        for r in range(1, _NS):
            acc = acc + histflat_v[pl.ds(r * 128 + j * _L, _L)]
        hist_v[0, pl.ds(j * _L, _L)] = acc
    pltpu.sync_copy(hist_v, hist_sh.at[pl.ds(sid, 1)])
    plsc.subcore_barrier()

    pltpu.sync_copy(hist_sh, tmph_v)
    for j in range(10):
        histpad_v[pl.ds(j * _L, _L)] = zeros
    for j in range(8):
        acc = tmph_v[0, pl.ds(j * _L, _L)]
        for r in range(1, _NS):
            acc = acc + tmph_v[r, pl.ds(j * _L, _L)]
        histpad_v[pl.ds(_L + j * _L, _L)] = acc

    for j in range(8):
        s = zeros
        for k in range(_KS):
            tap = plsc.load_gather(
                histpad_v, [_L + j * _L + (k - _half) + lanes])
            s = s + jnp.float32(_KW[k]) * tap
        sd_v[pl.ds(j * _L, _L)] = s

    acc = zeros
    for j in range(8):
        h = histpad_v[pl.ds(_L + j * _L, _L)]
        s = sd_v[pl.ds(j * _L, _L)]
        acc = acc + jnp.where(h > 0.0, h / s, 0.0)
    scaling = jnp.float32(_NL) / _xlane(acc, jnp.add)

    tb = (cid * _NS + sid) * _TC2
    pltpu.sync_copy(input_hbm.at[pl.ds(tb, _TC2)], inp_v)
    pltpu.sync_copy(target_hbm.at[pl.ds(tb, _TC2)], tgt_v)

    def tgt_body(i, carry):
        acc_a, acc_b, acc_w = carry
        tv = tgt_v[pl.ds(i * _L, _L)]
        iv = inp_v[pl.ds(i * _L, _L)]
        kw = plsc.load_gather(sd_v, [bidx(tv)])
        isz = kw == 0.0
        w = jnp.where(isz, zeros, 1.0 / kw)
        d = iv - tv
        loss = d * d
        acc_a = acc_a + jnp.where(isz, zeros, loss * w)
        acc_b = acc_b + jnp.where(isz, loss, zeros)
        acc_w = jnp.maximum(acc_w, jnp.where(isz, ninf, w))
        return acc_a, acc_b, acc_w

    acc_a, acc_b, acc_w = lax.fori_loop(
        0, _TC2 // _L, tgt_body, (zeros, zeros, ninf))

    xa = _xlane(acc_a, jnp.add)
    xb = _xlane(acc_b, jnp.add)
    xw = _xlane(acc_w, jnp.maximum)
    vals = jnp.where(lanes == 0, xa,
                     jnp.where(lanes == 1, xb,
                               jnp.where(lanes == 2, xw,
                                         jnp.where(lanes == 3, scaling,
                                                   zeros))))
    outrow_v[0, :] = vals
    pltpu.sync_copy(outrow_v, out_hbm.at[pl.ds(cid * _NS + sid, 1)])


@jax.jit
def _ldsloss(labels_pad, inp, tgt):
    mesh = plsc.VectorSubcoreMesh(core_axis_name="c", subcore_axis_name="s",
                                  num_cores=2, num_subcores=_NS)
    f = pl.kernel(
        _sc_body,
        out_type=jax.ShapeDtypeStruct((2 * _NS, _L), jnp.float32),
        mesh=mesh,
        compiler_params=pltpu.CompilerParams(needs_layout_passes=False),
        scratch_types=[
            pltpu.VMEM((_LCH,), jnp.float32),
            pltpu.VMEM((_TC2,), jnp.float32),
            pltpu.VMEM((_TC2,), jnp.float32),
            pltpu.VMEM((_NS * 128,), jnp.float32),
            pltpu.VMEM((1, 128), jnp.float32),
            pltpu.VMEM((160,), jnp.float32),
            pltpu.VMEM((128,), jnp.float32),
            pltpu.VMEM((144,), jnp.float32),
            pltpu.VMEM((2, _L), jnp.float32),
            pltpu.VMEM((_NS, _L), jnp.float32),
            pltpu.VMEM((_NS, _L), jnp.float32),
            pltpu.VMEM((_NS, 128), jnp.float32),
            pltpu.VMEM((1, _L), jnp.float32),
            pltpu.VMEM_SHARED((_NS, _L), jnp.float32),
            pltpu.VMEM_SHARED((_NS, _L), jnp.float32),
            pltpu.VMEM_SHARED((_NS, 128), jnp.float32),
        ],
    )
    rows = f(labels_pad, inp, tgt)
    av = jnp.sum(rows[:, 0])
    bv = jnp.sum(rows[:, 1])
    wv = jnp.max(rows[:, 2])
    scaling = rows[0, 3]
    wsafe = jnp.where(bv > 0.0, wv, jnp.float32(0.0))
    return (av + wsafe * bv) * scaling / jnp.float32(_NB)


def kernel(input, target, labels):
    labs = jnp.concatenate(
        [labels[:, 0], jnp.zeros((_NPAD - _NL,), jnp.float32)])
    return _ldsloss(labs, input[:, 0], target[:, 0])


# speedup vs baseline: 153.7217x; 1.0149x over previous
import functools

import jax
import jax.numpy as jnp
import numpy as np
from jax import lax
from jax.experimental import pallas as pl
from jax.experimental.pallas import tpu as pltpu
from jax.experimental.pallas import tpu_sc as plsc

_L = 16
_NS = 16
_NL = 100000
_NB = 16384
_LCH = 6256
_NPAD = _NS * _LCH
_TCH = _NB // _NS
_TC2 = _NB // (2 * _NS)
_KS = 9
_SIGMA = 1.0

_half = (_KS - 1) // 2
_x = np.arange(-_half, _half + 1, dtype=np.float64)
_kw = np.exp(-0.5 * (_x / _SIGMA) ** 2)
_KW = [float(v) for v in (_kw / _kw.max()).astype(np.float32)]


def _sc_body(labels_hbm, input_hbm, target_hbm, out_hbm,
             lab_v, inp_v, tgt_v, histflat_v, hist_v, histpad_v, sd_v,
             edges_v, stage_v, tmpa_v, tmpb_v, tmph_v, outrow_v,
             mm_min_sh, mm_max_sh, hist_sh):
    cid = lax.axis_index("c")
    sid = lax.axis_index("s")
    lanes = lax.broadcasted_iota(jnp.int32, (_L,), 0)

    def _xlane(v, op):
        s = v[0]
        for r in range(1, _L):
            s = op(s, v[r])
        return jnp.full((_L,), s)

    zeros = jnp.zeros((_L,), jnp.float32)
    ones = jnp.ones((_L,), jnp.float32)
    inf = jnp.full((_L,), jnp.inf, jnp.float32)
    ninf = jnp.full((_L,), -jnp.inf, jnp.float32)

    base = sid * _LCH
    pltpu.sync_copy(labels_hbm.at[pl.ds(base, _LCH)], lab_v)

    def mm_chunk(off, carry):
        mn, mx = carry
        v = lab_v[pl.ds(off, _L)]
        valid = (base + off + lanes) < _NL
        mn = jnp.minimum(mn, jnp.where(valid, v, inf))
        mx = jnp.maximum(mx, jnp.where(valid, v, ninf))
        return mn, mx

    _NCH = _LCH // _L
    _NC2 = _NCH // 2

    def mm_body(i, carry):
        carry = mm_chunk(2 * i * _L, carry)
        return mm_chunk((2 * i + 1) * _L, carry)

    mn, mx = lax.fori_loop(0, _NC2, mm_body, (inf, ninf))
    mn, mx = mm_chunk((_NCH - 1) * _L, (mn, mx))
    stage_v[0, :] = _xlane(mn, jnp.minimum)
    stage_v[1, :] = _xlane(mx, jnp.maximum)
    pltpu.sync_copy(stage_v.at[pl.ds(0, 1)], mm_min_sh.at[pl.ds(sid, 1)])
    pltpu.sync_copy(stage_v.at[pl.ds(1, 1)], mm_max_sh.at[pl.ds(sid, 1)])
    plsc.subcore_barrier()

    pltpu.sync_copy(mm_min_sh, tmpa_v)
    pltpu.sync_copy(mm_max_sh, tmpb_v)
    gmin = tmpa_v[0, :]
    gmax = tmpb_v[0, :]
    for r in range(1, _NS):
        gmin = jnp.minimum(gmin, tmpa_v[r, :])
        gmax = jnp.maximum(gmax, tmpb_v[r, :])

    nbi = ((gmax - gmin) / jnp.float32(0.1)).astype(jnp.int32)
    nbf = nbi.astype(jnp.float32)
    invr = nbf / (gmax - gmin)

    for j in range(9):
        jv = j * _L + lanes
        step = jv.astype(jnp.float32) / nbf
        e = gmin * (1.0 - step) + gmax * step
        e = jnp.where(jv == nbi, gmax, e)
        e = jnp.where(jv > nbi, inf, e)
        edges_v[pl.ds(j * _L, _L)] = e

    def bidx(v):
        c = jnp.clip(((v - gmin) * invr).astype(jnp.int32), 0, nbi - 1)
        e0 = plsc.load_gather(edges_v, [c])
        e1 = plsc.load_gather(edges_v, [c + 1])
        t = c - (e0 > v).astype(jnp.int32) + (e1 <= v).astype(jnp.int32)
        return jnp.where(t < 0, nbi - 1, jnp.minimum(t, nbi - 1))

    def zero_body(i, _):
        histflat_v[pl.ds(i * _L, _L)] = zeros
        return 0

    lax.fori_loop(0, (_NS * 128) // _L, zero_body, 0)

    def hist_chunk(off):
        v = lab_v[pl.ds(off, _L)]
        valid = (base + off + lanes) < _NL
        idx = bidx(v)
        plsc.addupdate_scatter(histflat_v, [lanes * 128 + idx], ones,
                               mask=valid)

    def hist_body(i, _):
        hist_chunk(2 * i * _L)
        hist_chunk((2 * i + 1) * _L)
        return 0

    lax.fori_loop(0, _NC2, hist_body, 0)
    hist_chunk((_NCH - 1) * _L)

    for j in range(8):
        acc = histflat_v[pl.ds(j * _L, _L)]
        for r in range(1, _NS):
            acc = acc + histflat_v[pl.ds(r * 128 + j * _L, _L)]
        hist_v[0, pl.ds(j * _L, _L)] = acc
    pltpu.sync_copy(hist_v, hist_sh.at[pl.ds(sid, 1)])
    plsc.subcore_barrier()

    pltpu.sync_copy(hist_sh, tmph_v)
    for j in range(10):
        histpad_v[pl.ds(j * _L, _L)] = zeros
    for j in range(8):
        acc = tmph_v[0, pl.ds(j * _L, _L)]
        for r in range(1, _NS):
            acc = acc + tmph_v[r, pl.ds(j * _L, _L)]
        histpad_v[pl.ds(_L + j * _L, _L)] = acc

    for j in range(8):
        s = zeros
        for k in range(_KS):
            tap = plsc.load_gather(
                histpad_v, [_L + j * _L + (k - _half) + lanes])
            s = s + jnp.float32(_KW[k]) * tap
        sd_v[pl.ds(j * _L, _L)] = s

    acc = zeros
    for j in range(8):
        h = histpad_v[pl.ds(_L + j * _L, _L)]
        s = sd_v[pl.ds(j * _L, _L)]
        acc = acc + jnp.where(h > 0.0, h / s, 0.0)
    scaling = jnp.float32(_NL) / _xlane(acc, jnp.add)

    tb = (cid * _NS + sid) * _TC2
    pltpu.sync_copy(input_hbm.at[pl.ds(tb, _TC2)], inp_v)
    pltpu.sync_copy(target_hbm.at[pl.ds(tb, _TC2)], tgt_v)

    def tgt_body(i, carry):
        acc_a, acc_b, acc_w = carry
        tv = tgt_v[pl.ds(i * _L, _L)]
        iv = inp_v[pl.ds(i * _L, _L)]
        kw = plsc.load_gather(sd_v, [bidx(tv)])
        isz = kw == 0.0
        w = jnp.where(isz, zeros, 1.0 / kw)
        d = iv - tv
        loss = d * d
        acc_a = acc_a + jnp.where(isz, zeros, loss * w)
        acc_b = acc_b + jnp.where(isz, loss, zeros)
        acc_w = jnp.maximum(acc_w, jnp.where(isz, ninf, w))
        return acc_a, acc_b, acc_w

    acc_a, acc_b, acc_w = lax.fori_loop(
        0, _TC2 // _L, tgt_body, (zeros, zeros, ninf))

    xa = _xlane(acc_a, jnp.add)
    xb = _xlane(acc_b, jnp.add)
    xw = _xlane(acc_w, jnp.maximum)
    vals = jnp.where(lanes == 0, xa,
                     jnp.where(lanes == 1, xb,
                               jnp.where(lanes == 2, xw,
                                         jnp.where(lanes == 3, scaling,
                                                   zeros))))
    outrow_v[0, :] = vals
    pltpu.sync_copy(outrow_v, out_hbm.at[pl.ds(cid * _NS + sid, 1)])


@jax.jit
def _ldsloss(labels_pad, inp, tgt):
    mesh = plsc.VectorSubcoreMesh(core_axis_name="c", subcore_axis_name="s",
                                  num_cores=2, num_subcores=_NS)
    f = pl.kernel(
        _sc_body,
        out_type=jax.ShapeDtypeStruct((2 * _NS, _L), jnp.float32),
        mesh=mesh,
        compiler_params=pltpu.CompilerParams(needs_layout_passes=False),
        scratch_types=[
            pltpu.VMEM((_LCH,), jnp.float32),
            pltpu.VMEM((_TC2,), jnp.float32),
            pltpu.VMEM((_TC2,), jnp.float32),
            pltpu.VMEM((_NS * 128,), jnp.float32),
            pltpu.VMEM((1, 128), jnp.float32),
            pltpu.VMEM((160,), jnp.float32),
            pltpu.VMEM((128,), jnp.float32),
            pltpu.VMEM((144,), jnp.float32),
            pltpu.VMEM((2, _L), jnp.float32),
            pltpu.VMEM((_NS, _L), jnp.float32),
            pltpu.VMEM((_NS, _L), jnp.float32),
            pltpu.VMEM((_NS, 128), jnp.float32),
            pltpu.VMEM((1, _L), jnp.float32),
            pltpu.VMEM_SHARED((_NS, _L), jnp.float32),
            pltpu.VMEM_SHARED((_NS, _L), jnp.float32),
            pltpu.VMEM_SHARED((_NS, 128), jnp.float32),
        ],
    )
    rows = f(labels_pad, inp, tgt)
    av = jnp.sum(rows[:, 0])
    bv = jnp.sum(rows[:, 1])
    wv = jnp.max(rows[:, 2])
    scaling = rows[0, 3]
    wsafe = jnp.where(bv > 0.0, wv, jnp.float32(0.0))
    return (av + wsafe * bv) * scaling / jnp.float32(_NB)


def kernel(input, target, labels):
    labs = jnp.concatenate(
        [labels[:, 0], jnp.zeros((_NPAD - _NL,), jnp.float32)])
    return _ldsloss(labs, input[:, 0], target[:, 0])
